# Initial kernel scaffold; baseline (speedup 1.0000x reference)
import jax
import jax.numpy as jnp
from jax.experimental import pallas as pl


def kernel(x, batch, table):
    raise NotImplementedError("write your pallas kernel here")


# speedup vs baseline: 2.0490x; 2.0490x over previous
import jax
import jax.numpy as jnp
from jax import lax
from jax.experimental import pallas as pl
from jax.experimental.pallas import tpu as pltpu
from jax.experimental.pallas import tpu_sc as plsc

N = 100000
DIM = 128
TABLE_ROWS = 4096
NC = 2
NS = 16
NW = NC * NS
L = 16

CHUNK = 128
N_FULL = N // CHUNK
TAIL = N - N_FULL * CHUNK
PER_W = N_FULL // NW
REM = N_FULL % NW

SCAN = (-(-N // NS) + L - 1) // L * L
SCAN_LAST = N - (NS - 1) * SCAN
assert SCAN % 16 == 0 and SCAN_LAST % 16 == 0 and SCAN_LAST > 0


def _body(x_hbm, batch_hbm, table_hbm, out_hbm,
          scanbuf, local64, iota64, zeros64, starts_v, shared64,
          bbuf, ibuf, xbuf, rbuf):
    cid = lax.axis_index("c")
    sid = lax.axis_index("s")
    iota16 = lax.iota(jnp.int32, L)

    for t in range(4):
        z = jnp.zeros((L,), jnp.int32)
        local64[pl.ds(L * t, L)] = z
        zeros64[pl.ds(L * t, L)] = z
        iota64[pl.ds(L * t, L)] = iota16 + L * t

    base = sid * SCAN

    @pl.when(sid == 0)
    def _():
        scanbuf[pl.ds(0, L)] = jnp.full((L,), -1, jnp.int32)
        pltpu.sync_copy(batch_hbm.at[pl.ds(0, SCAN)],
                        scanbuf.at[pl.ds(L, SCAN)])

    @pl.when(jnp.logical_and(sid > 0, sid < NS - 1))
    def _():
        pltpu.sync_copy(batch_hbm.at[pl.ds(base - L, SCAN + L)],
                        scanbuf.at[pl.ds(0, SCAN + L)])

    @pl.when(sid == NS - 1)
    def _():
        pltpu.sync_copy(batch_hbm.at[pl.ds(base - L, SCAN_LAST + L)],
                        scanbuf.at[pl.ds(0, SCAN_LAST + L)])

    trip = jnp.where(sid == NS - 1, SCAN_LAST // L, SCAN // L)

    def scan_step(j, carry):
        off = j * L
        vprev = scanbuf[pl.ds(off + L - 1, L)]
        vcur = scanbuf[pl.ds(off + L, L)]
        jump = vprev != vcur
        g = base + off + iota16
        plsc.store_scatter(local64, [vcur], g + 1, mask=jump)
        return carry

    lax.fori_loop(0, trip, scan_step, 0)

    @pl.when(sid == 0)
    def _():
        pltpu.sync_copy(zeros64, shared64)
    plsc.subcore_barrier()
    pltpu.sync_copy(local64, shared64.at[iota64], add=True)
    plsc.subcore_barrier()
    pltpu.sync_copy(shared64, starts_v)

    wid = sid * NC + cid

    def do_rows(base_row, nrows):
        pltpu.sync_copy(batch_hbm.at[pl.ds(base_row, nrows)],
                        bbuf.at[pl.ds(0, nrows)])
        pltpu.sync_copy(x_hbm.at[pl.ds(base_row, nrows)],
                        xbuf.at[pl.ds(0, nrows)])
        for j in range(nrows // L):
            vb = bbuf[pl.ds(L * j, L)]
            st = plsc.load_gather(starts_v, [vb])
            pos = base_row + L * j + iota16 - st + 1
            ibuf[pl.ds(L * j, L)] = jnp.minimum(pos, TABLE_ROWS - 1)
        if nrows == CHUNK:
            pltpu.sync_copy(table_hbm.at[ibuf], rbuf)
        else:
            pltpu.sync_copy(table_hbm.at[ibuf.at[pl.ds(0, nrows)]],
                            rbuf.at[pl.ds(0, nrows)])

        def add_row(r, carry):
            for k in range(DIM // L):
                sl = pl.ds(L * k, L)
                xbuf[r, sl] = xbuf[r, sl] + rbuf[r, sl]
            return carry

        lax.fori_loop(0, nrows, add_row, 0)
        pltpu.sync_copy(xbuf.at[pl.ds(0, nrows)],
                        out_hbm.at[pl.ds(base_row, nrows)])

    nchunks = PER_W + jnp.where(wid < REM, 1, 0)

    def chunk_step(k, carry):
        c = wid + k * NW
        do_rows(c * CHUNK, CHUNK)
        return carry

    lax.fori_loop(0, nchunks, chunk_step, 0)

    @pl.when(wid == NW - 1)
    def _():
        do_rows(N_FULL * CHUNK, TAIL)


def kernel(x, batch, table):
    mesh = plsc.VectorSubcoreMesh(core_axis_name="c", subcore_axis_name="s")
    f = pl.kernel(
        _body,
        out_type=jax.ShapeDtypeStruct((N, DIM), jnp.float32),
        mesh=mesh,
        compiler_params=pltpu.CompilerParams(needs_layout_passes=False),
        scratch_types=[
            pltpu.VMEM((SCAN + L,), jnp.int32),
            pltpu.VMEM((64,), jnp.int32),
            pltpu.VMEM((64,), jnp.int32),
            pltpu.VMEM((64,), jnp.int32),
            pltpu.VMEM((64,), jnp.int32),
            pltpu.VMEM_SHARED((64,), jnp.int32),
            pltpu.VMEM((CHUNK,), jnp.int32),
            pltpu.VMEM((CHUNK,), jnp.int32),
            pltpu.VMEM((CHUNK, DIM), jnp.float32),
            pltpu.VMEM((CHUNK, DIM), jnp.float32),
        ],
    )
    return f(x, batch.astype(jnp.int32), table)


# speedup vs baseline: 2.8906x; 1.4108x over previous
import jax
import jax.numpy as jnp
from jax import lax
from jax.experimental import pallas as pl
from jax.experimental.pallas import tpu as pltpu
from jax.experimental.pallas import tpu_sc as plsc

N = 100000
DIM = 128
TABLE_ROWS = 4096
NC = 2
NS = 16
NW = NC * NS
L = 16

CHUNK = 128
N_FULL = N // CHUNK
TAIL = N - N_FULL * CHUNK
PER_W = N_FULL // NW
REM = N_FULL % NW

SCAN = (-(-N // NS) + L - 1) // L * L
SCAN_LAST = N - (NS - 1) * SCAN
assert SCAN % 16 == 0 and SCAN_LAST % 16 == 0 and SCAN_LAST > 0


def _body(x_hbm, batch_hbm, table_hbm, out_hbm,
          scanbuf, local64, iota64, zeros64, starts_v, shared64,
          bb0, bb1, ib0, ib1, xb0, xb1, rb0, rb1,
          sx0, sx1, sb0, sb1, sg0, sg1, ss0, ss1):
    cid = lax.axis_index("c")
    sid = lax.axis_index("s")
    iota16 = lax.iota(jnp.int32, L)

    for t in range(4):
        z = jnp.zeros((L,), jnp.int32)
        local64[pl.ds(L * t, L)] = z
        zeros64[pl.ds(L * t, L)] = z
        iota64[pl.ds(L * t, L)] = iota16 + L * t

    base = sid * SCAN

    @pl.when(sid == 0)
    def _():
        scanbuf[pl.ds(0, L)] = jnp.full((L,), -1, jnp.int32)
        pltpu.sync_copy(batch_hbm.at[pl.ds(0, SCAN)],
                        scanbuf.at[pl.ds(L, SCAN)])

    @pl.when(jnp.logical_and(sid > 0, sid < NS - 1))
    def _():
        pltpu.sync_copy(batch_hbm.at[pl.ds(base - L, SCAN + L)],
                        scanbuf.at[pl.ds(0, SCAN + L)])

    @pl.when(sid == NS - 1)
    def _():
        pltpu.sync_copy(batch_hbm.at[pl.ds(base - L, SCAN_LAST + L)],
                        scanbuf.at[pl.ds(0, SCAN_LAST + L)])

    trip = jnp.where(sid == NS - 1, SCAN_LAST // L, SCAN // L)

    def scan_step(j, carry):
        off = j * L
        vprev = scanbuf[pl.ds(off + L - 1, L)]
        vcur = scanbuf[pl.ds(off + L, L)]
        jump = vprev != vcur
        g = base + off + iota16
        plsc.store_scatter(local64, [vcur], g + 1, mask=jump)
        return carry

    lax.fori_loop(0, trip, scan_step, 0)

    @pl.when(sid == 0)
    def _():
        pltpu.sync_copy(zeros64, shared64)
    plsc.subcore_barrier()
    pltpu.sync_copy(local64, shared64.at[iota64], add=True)
    plsc.subcore_barrier()
    pltpu.sync_copy(shared64, starts_v)

    wid = sid * NC + cid
    nchunks = PER_W + jnp.where(wid < REM, 1, 0)
    bufs = ((bb0, ib0, xb0, rb0, sx0, sb0, sg0, ss0),
            (bb1, ib1, xb1, rb1, sx1, sb1, sg1, ss1))

    def cbase(k):
        return (wid + k * NW) * CHUNK

    def issue_loads(k, B):
        bb, ib, xb, rb, sx, sbm, sg, ss = B
        pltpu.async_copy(batch_hbm.at[pl.ds(cbase(k), CHUNK)], bb, sbm)
        pltpu.async_copy(x_hbm.at[pl.ds(cbase(k), CHUNK)], xb, sx)

    def wait_batch(k, B):
        bb, ib, xb, rb, sx, sbm, sg, ss = B
        pltpu.make_async_copy(batch_hbm.at[pl.ds(cbase(k), CHUNK)],
                              bb, sbm).wait()

    def compute_idx(k, B):
        bb, ib, xb, rb, sx, sbm, sg, ss = B
        b0 = cbase(k)
        for j in range(CHUNK // L):
            vb = bb[pl.ds(L * j, L)]
            st = plsc.load_gather(starts_v, [vb])
            pos = b0 + L * j + iota16 - st + 1
            ib[pl.ds(L * j, L)] = jnp.minimum(pos, TABLE_ROWS - 1)

    def issue_gather(k, B):
        bb, ib, xb, rb, sx, sbm, sg, ss = B
        pltpu.async_copy(table_hbm.at[ib], rb, sg)

    def wait_x_gather(k, B):
        bb, ib, xb, rb, sx, sbm, sg, ss = B
        pltpu.make_async_copy(x_hbm.at[pl.ds(cbase(k), CHUNK)],
                              xb, sx).wait()
        pltpu.make_async_copy(table_hbm.at[ib], rb, sg).wait()

    def accumulate(B):
        bb, ib, xb, rb, sx, sbm, sg, ss = B

        def add_row(r, carry):
            for t in range(DIM // L):
                sl = pl.ds(L * t, L)
                plsc.addupdate(rb.at[r, sl], xb[r, sl])
            return carry

        lax.fori_loop(0, CHUNK, add_row, 0)

    def issue_store(k, B):
        bb, ib, xb, rb, sx, sbm, sg, ss = B
        pltpu.async_copy(rb, out_hbm.at[pl.ds(cbase(k), CHUNK)], ss)

    def wait_store(k, B):
        bb, ib, xb, rb, sx, sbm, sg, ss = B
        pltpu.make_async_copy(rb, out_hbm.at[pl.ds(cbase(k), CHUNK)],
                              ss).wait()

    @pl.when(wid == NW - 1)
    def _():
        tb = N_FULL * CHUNK
        pltpu.sync_copy(batch_hbm.at[pl.ds(tb, TAIL)], bb0.at[pl.ds(0, TAIL)])
        pltpu.sync_copy(x_hbm.at[pl.ds(tb, TAIL)], xb0.at[pl.ds(0, TAIL)])
        for j in range(TAIL // L):
            vb = bb0[pl.ds(L * j, L)]
            st = plsc.load_gather(starts_v, [vb])
            pos = tb + L * j + iota16 - st + 1
            ib0[pl.ds(L * j, L)] = jnp.minimum(pos, TABLE_ROWS - 1)
        pltpu.sync_copy(table_hbm.at[ib0.at[pl.ds(0, TAIL)]],
                        rb0.at[pl.ds(0, TAIL)])

        def add_row(r, carry):
            for t in range(DIM // L):
                sl = pl.ds(L * t, L)
                plsc.addupdate(rb0.at[r, sl], xb0[r, sl])
            return carry

        lax.fori_loop(0, TAIL, add_row, 0)
        pltpu.sync_copy(rb0.at[pl.ds(0, TAIL)], out_hbm.at[pl.ds(tb, TAIL)])

    n = nchunks

    issue_loads(0, bufs[0])
    issue_loads(1, bufs[1])
    wait_batch(0, bufs[0])
    compute_idx(0, bufs[0])
    issue_gather(0, bufs[0])

    def iter_body(k, B, Bo):
        wait_x_gather(k, B)
        accumulate(B)
        issue_store(k, B)

        @pl.when(k + 2 < n)
        def _():
            issue_loads(k + 2, B)

        @pl.when(k + 1 < n)
        def _():
            wait_batch(k + 1, Bo)
            compute_idx(k + 1, Bo)

            @pl.when(k >= 1)
            def _():
                wait_store(k - 1, Bo)

            issue_gather(k + 1, Bo)

    def loop_body(k2, carry):
        k = 2 * k2

        @pl.when(k < n)
        def _():
            iter_body(k, bufs[0], bufs[1])

        @pl.when(k + 1 < n)
        def _():
            iter_body(k + 1, bufs[1], bufs[0])

        return carry

    lax.fori_loop(0, (PER_W + 2) // 2, loop_body, 0)

    @pl.when(wid < REM)
    def _():
        wait_store(PER_W, bufs[PER_W % 2])
        wait_store(PER_W - 1, bufs[(PER_W - 1) % 2])

    @pl.when(wid >= REM)
    def _():
        wait_store(PER_W - 1, bufs[(PER_W - 1) % 2])
        wait_store(PER_W - 2, bufs[(PER_W - 2) % 2])


def kernel(x, batch, table):
    mesh = plsc.VectorSubcoreMesh(core_axis_name="c", subcore_axis_name="s")
    f = pl.kernel(
        _body,
        out_type=jax.ShapeDtypeStruct((N, DIM), jnp.float32),
        mesh=mesh,
        compiler_params=pltpu.CompilerParams(needs_layout_passes=False),
        scratch_types=[
            pltpu.VMEM((SCAN + L,), jnp.int32),
            pltpu.VMEM((64,), jnp.int32),
            pltpu.VMEM((64,), jnp.int32),
            pltpu.VMEM((64,), jnp.int32),
            pltpu.VMEM((64,), jnp.int32),
            pltpu.VMEM_SHARED((64,), jnp.int32),
            pltpu.VMEM((CHUNK,), jnp.int32),
            pltpu.VMEM((CHUNK,), jnp.int32),
            pltpu.VMEM((CHUNK,), jnp.int32),
            pltpu.VMEM((CHUNK,), jnp.int32),
            pltpu.VMEM((CHUNK, DIM), jnp.float32),
            pltpu.VMEM((CHUNK, DIM), jnp.float32),
            pltpu.VMEM((CHUNK, DIM), jnp.float32),
            pltpu.VMEM((CHUNK, DIM), jnp.float32),
            pltpu.SemaphoreType.DMA,
            pltpu.SemaphoreType.DMA,
            pltpu.SemaphoreType.DMA,
            pltpu.SemaphoreType.DMA,
            pltpu.SemaphoreType.DMA,
            pltpu.SemaphoreType.DMA,
            pltpu.SemaphoreType.DMA,
            pltpu.SemaphoreType.DMA,
        ],
    )
    return f(x, batch.astype(jnp.int32), table)


# speedup vs baseline: 3.4909x; 1.2077x over previous
import jax
import jax.numpy as jnp
from jax import lax
from jax.experimental import pallas as pl
from jax.experimental.pallas import tpu as pltpu
from jax.experimental.pallas import tpu_sc as plsc

N = 100000
DIM = 128
TABLE_ROWS = 4096
NC = 2
NS = 16
NW = NC * NS
L = 16

CHUNK = 128
N_FULL = N // CHUNK
TAIL = N - N_FULL * CHUNK
PER_W = N_FULL // NW
REM = N_FULL % NW

SCAN = (-(-N // NS) + L - 1) // L * L
SCAN_LAST = N - (NS - 1) * SCAN
assert SCAN % 16 == 0 and SCAN_LAST % 16 == 0 and SCAN_LAST > 0

NXB = 4


def _body(x_hbm, batch_hbm, table_hbm, out_hbm,
          scanbuf, local64, iota64, zeros64, starts_v, shared64,
          bb0, bb1, ib0, ib1, xb0, xb1, xb2, xb3,
          sb0, sb1, sx0, sx1, sx2, sx3,
          sg0, sg1, sg2, sg3, ss0, ss1, ss2, ss3):
    cid = lax.axis_index("c")
    sid = lax.axis_index("s")
    iota16 = lax.iota(jnp.int32, L)

    for t in range(4):
        z = jnp.zeros((L,), jnp.int32)
        local64[pl.ds(L * t, L)] = z
        zeros64[pl.ds(L * t, L)] = z
        iota64[pl.ds(L * t, L)] = iota16 + L * t

    base = sid * SCAN

    @pl.when(sid == 0)
    def _():
        scanbuf[pl.ds(0, L)] = jnp.full((L,), -1, jnp.int32)
        pltpu.sync_copy(batch_hbm.at[pl.ds(0, SCAN)],
                        scanbuf.at[pl.ds(L, SCAN)])

    @pl.when(jnp.logical_and(sid > 0, sid < NS - 1))
    def _():
        pltpu.sync_copy(batch_hbm.at[pl.ds(base - L, SCAN + L)],
                        scanbuf.at[pl.ds(0, SCAN + L)])

    @pl.when(sid == NS - 1)
    def _():
        pltpu.sync_copy(batch_hbm.at[pl.ds(base - L, SCAN_LAST + L)],
                        scanbuf.at[pl.ds(0, SCAN_LAST + L)])

    trip = jnp.where(sid == NS - 1, SCAN_LAST // L, SCAN // L)

    def scan_step(j, carry):
        off = j * L
        vprev = scanbuf[pl.ds(off + L - 1, L)]
        vcur = scanbuf[pl.ds(off + L, L)]
        jump = vprev != vcur
        g = base + off + iota16
        plsc.store_scatter(local64, [vcur], g + 1, mask=jump)
        return carry

    lax.fori_loop(0, trip, scan_step, 0)

    @pl.when(sid == 0)
    def _():
        pltpu.sync_copy(zeros64, shared64)
    plsc.subcore_barrier()
    pltpu.sync_copy(local64, shared64.at[iota64], add=True)
    plsc.subcore_barrier()
    pltpu.sync_copy(shared64, starts_v)

    wid = sid * NC + cid
    nchunks = PER_W + jnp.where(wid < REM, 1, 0)
    SB = ((bb0, ib0, sb0), (bb1, ib1, sb1))
    XB = ((xb0, sx0, sg0, ss0), (xb1, sx1, sg1, ss1),
          (xb2, sx2, sg2, ss2), (xb3, sx3, sg3, ss3))

    def cbase(k):
        return (wid + k * NW) * CHUNK

    def issue_loads(k, S, X):
        bb, ib, sbm = S
        xb, sx, sg, ss = X
        pltpu.async_copy(batch_hbm.at[pl.ds(cbase(k), CHUNK)], bb, sbm)
        pltpu.async_copy(x_hbm.at[pl.ds(cbase(k), CHUNK)], xb, sx)

    def compute_idx(k, S):
        bb, ib, sbm = S
        pltpu.make_async_copy(batch_hbm.at[pl.ds(cbase(k), CHUNK)],
                              bb, sbm).wait()
        b0 = cbase(k)
        for j in range(CHUNK // L):
            vb = bb[pl.ds(L * j, L)]
            st = plsc.load_gather(starts_v, [vb])
            pos = b0 + L * j + iota16 - st + 1
            ib[pl.ds(L * j, L)] = jnp.minimum(pos, TABLE_ROWS - 1)

    def issue_gather_add(k, S, X):
        bb, ib, sbm = S
        xb, sx, sg, ss = X
        pltpu.make_async_copy(x_hbm.at[pl.ds(cbase(k), CHUNK)], xb, sx).wait()
        pltpu.async_copy(table_hbm.at[ib], xb, sg, add=True)

    def wait_gather(k, S, X):
        bb, ib, sbm = S
        xb, sx, sg, ss = X
        pltpu.make_async_copy(table_hbm.at[ib], xb, sg).wait()

    def issue_store(k, X):
        xb, sx, sg, ss = X
        pltpu.async_copy(xb, out_hbm.at[pl.ds(cbase(k), CHUNK)], ss)

    def wait_store(k, X):
        xb, sx, sg, ss = X
        pltpu.make_async_copy(xb, out_hbm.at[pl.ds(cbase(k), CHUNK)],
                              ss).wait()

    @pl.when(wid == NW - 1)
    def _():
        tb = N_FULL * CHUNK
        pltpu.sync_copy(batch_hbm.at[pl.ds(tb, TAIL)], bb0.at[pl.ds(0, TAIL)])
        pltpu.sync_copy(x_hbm.at[pl.ds(tb, TAIL)], xb0.at[pl.ds(0, TAIL)])
        for j in range(TAIL // L):
            vb = bb0[pl.ds(L * j, L)]
            st = plsc.load_gather(starts_v, [vb])
            pos = tb + L * j + iota16 - st + 1
            ib0[pl.ds(L * j, L)] = jnp.minimum(pos, TABLE_ROWS - 1)
        pltpu.async_copy(table_hbm.at[ib0.at[pl.ds(0, TAIL)]],
                         xb0.at[pl.ds(0, TAIL)], sg0, add=True).wait()
        pltpu.sync_copy(xb0.at[pl.ds(0, TAIL)], out_hbm.at[pl.ds(tb, TAIL)])

    n = nchunks

    issue_loads(0, SB[0], XB[0])
    issue_loads(1, SB[1], XB[1])
    compute_idx(0, SB[0])
    issue_gather_add(0, SB[0], XB[0])

    def iter_body(k, b2, b4):
        wait_gather(k, SB[b2], XB[b4])
        issue_store(k, XB[b4])

        @pl.when(k + 2 < n)
        def _():
            @pl.when(k >= 2)
            def _():
                wait_store(k - 2, XB[(b4 + 2) % NXB])
            issue_loads(k + 2, SB[b2], XB[(b4 + 2) % NXB])

        @pl.when(k + 1 < n)
        def _():
            compute_idx(k + 1, SB[1 - b2])
            issue_gather_add(k + 1, SB[1 - b2], XB[(b4 + 1) % NXB])

    def loop_body(k4, carry):
        for off in range(NXB):
            k = NXB * k4 + off

            @pl.when(k < n)
            def _():
                iter_body(k, off % 2, off)

        return carry

    lax.fori_loop(0, (PER_W + NXB) // NXB, loop_body, 0)

    @pl.when(wid < REM)
    def _():
        for k in range(PER_W - 3, PER_W + 1):
            wait_store(k, XB[k % NXB])

    @pl.when(wid >= REM)
    def _():
        for k in range(PER_W - 4, PER_W):
            wait_store(k, XB[k % NXB])


def kernel(x, batch, table):
    mesh = plsc.VectorSubcoreMesh(core_axis_name="c", subcore_axis_name="s")
    f = pl.kernel(
        _body,
        out_type=jax.ShapeDtypeStruct((N, DIM), jnp.float32),
        mesh=mesh,
        compiler_params=pltpu.CompilerParams(needs_layout_passes=False),
        scratch_types=[
            pltpu.VMEM((SCAN + L,), jnp.int32),
            pltpu.VMEM((64,), jnp.int32),
            pltpu.VMEM((64,), jnp.int32),
            pltpu.VMEM((64,), jnp.int32),
            pltpu.VMEM((64,), jnp.int32),
            pltpu.VMEM_SHARED((64,), jnp.int32),
            pltpu.VMEM((CHUNK,), jnp.int32),
            pltpu.VMEM((CHUNK,), jnp.int32),
            pltpu.VMEM((CHUNK,), jnp.int32),
            pltpu.VMEM((CHUNK,), jnp.int32),
            pltpu.VMEM((CHUNK, DIM), jnp.float32),
            pltpu.VMEM((CHUNK, DIM), jnp.float32),
            pltpu.VMEM((CHUNK, DIM), jnp.float32),
            pltpu.VMEM((CHUNK, DIM), jnp.float32),
            pltpu.SemaphoreType.DMA,
            pltpu.SemaphoreType.DMA,
            pltpu.SemaphoreType.DMA,
            pltpu.SemaphoreType.DMA,
            pltpu.SemaphoreType.DMA,
            pltpu.SemaphoreType.DMA,
            pltpu.SemaphoreType.DMA,
            pltpu.SemaphoreType.DMA,
            pltpu.SemaphoreType.DMA,
            pltpu.SemaphoreType.DMA,
            pltpu.SemaphoreType.DMA,
            pltpu.SemaphoreType.DMA,
            pltpu.SemaphoreType.DMA,
            pltpu.SemaphoreType.DMA,
        ],
    )
    return f(x, batch.astype(jnp.int32), table)


# speedup vs baseline: 4.8379x; 1.3858x over previous
import jax
import jax.numpy as jnp
from jax import lax
from jax.experimental import pallas as pl
from jax.experimental.pallas import tpu as pltpu
from jax.experimental.pallas import tpu_sc as plsc

N = 100000
DIM = 128
TABLE_ROWS = 4096
NC = 2
NS = 16
NW = NC * NS
L = 16

CHUNK = 128
N_FULL = N // CHUNK
TAIL = N - N_FULL * CHUNK
PER_W = N_FULL // NW
REM = N_FULL % NW

SCAN = (-(-N // NS) + L - 1) // L * L
SCAN_LAST = N - (NS - 1) * SCAN
assert SCAN % 16 == 0 and SCAN_LAST % 16 == 0 and SCAN_LAST > 0

NXB = 4
TROWS_PER_TILE = TABLE_ROWS // NS


def _body(x_hbm, batch_hbm, table_hbm, out_hbm,
          scanbuf, local64, iota64, zeros64, starts_v, shared64,
          shared_tab, bb0, bb1, ib0, ib1, xb0, xb1, xb2, xb3,
          sb0, sb1, sx0, sx1, sx2, sx3,
          sg0, sg1, sg2, sg3, ss0, ss1, ss2, ss3, st0):
    cid = lax.axis_index("c")
    sid = lax.axis_index("s")
    iota16 = lax.iota(jnp.int32, L)

    pltpu.async_copy(table_hbm.at[pl.ds(sid * TROWS_PER_TILE, TROWS_PER_TILE)],
                     shared_tab.at[pl.ds(sid * TROWS_PER_TILE, TROWS_PER_TILE)],
                     st0)

    for t in range(4):
        z = jnp.zeros((L,), jnp.int32)
        local64[pl.ds(L * t, L)] = z
        zeros64[pl.ds(L * t, L)] = z
        iota64[pl.ds(L * t, L)] = iota16 + L * t

    base = sid * SCAN

    @pl.when(sid == 0)
    def _():
        scanbuf[pl.ds(0, L)] = jnp.full((L,), -1, jnp.int32)
        pltpu.sync_copy(batch_hbm.at[pl.ds(0, SCAN)],
                        scanbuf.at[pl.ds(L, SCAN)])

    @pl.when(jnp.logical_and(sid > 0, sid < NS - 1))
    def _():
        pltpu.sync_copy(batch_hbm.at[pl.ds(base - L, SCAN + L)],
                        scanbuf.at[pl.ds(0, SCAN + L)])

    @pl.when(sid == NS - 1)
    def _():
        pltpu.sync_copy(batch_hbm.at[pl.ds(base - L, SCAN_LAST + L)],
                        scanbuf.at[pl.ds(0, SCAN_LAST + L)])

    trip = jnp.where(sid == NS - 1, SCAN_LAST // L, SCAN // L)

    def scan_step(j, carry):
        off = j * L
        vprev = scanbuf[pl.ds(off + L - 1, L)]
        vcur = scanbuf[pl.ds(off + L, L)]
        jump = vprev != vcur
        g = base + off + iota16
        plsc.store_scatter(local64, [vcur], g + 1, mask=jump)
        return carry

    lax.fori_loop(0, trip, scan_step, 0)

    @pl.when(sid == 0)
    def _():
        pltpu.sync_copy(zeros64, shared64)
    plsc.subcore_barrier()
    pltpu.sync_copy(local64, shared64.at[iota64], add=True)
    pltpu.make_async_copy(
        table_hbm.at[pl.ds(sid * TROWS_PER_TILE, TROWS_PER_TILE)],
        shared_tab.at[pl.ds(sid * TROWS_PER_TILE, TROWS_PER_TILE)],
        st0).wait()
    plsc.subcore_barrier()
    pltpu.sync_copy(shared64, starts_v)

    wid = sid * NC + cid
    nchunks = PER_W + jnp.where(wid < REM, 1, 0)
    SB = ((bb0, ib0, sb0), (bb1, ib1, sb1))
    XB = ((xb0, sx0, sg0, ss0), (xb1, sx1, sg1, ss1),
          (xb2, sx2, sg2, ss2), (xb3, sx3, sg3, ss3))

    def cbase(k):
        return (wid + k * NW) * CHUNK

    def issue_loads(k, S, X):
        bb, ib, sbm = S
        xb, sx, sg, ss = X
        pltpu.async_copy(batch_hbm.at[pl.ds(cbase(k), CHUNK)], bb, sbm)
        pltpu.async_copy(x_hbm.at[pl.ds(cbase(k), CHUNK)], xb, sx)

    def compute_idx(k, S):
        bb, ib, sbm = S
        pltpu.make_async_copy(batch_hbm.at[pl.ds(cbase(k), CHUNK)],
                              bb, sbm).wait()
        b0 = cbase(k)
        for j in range(CHUNK // L):
            vb = bb[pl.ds(L * j, L)]
            st = plsc.load_gather(starts_v, [vb])
            pos = b0 + L * j + iota16 - st + 1
            ib[pl.ds(L * j, L)] = jnp.minimum(pos, TABLE_ROWS - 1)

    def issue_gather_add(k, S, X):
        bb, ib, sbm = S
        xb, sx, sg, ss = X
        pltpu.make_async_copy(x_hbm.at[pl.ds(cbase(k), CHUNK)], xb, sx).wait()
        pltpu.async_copy(shared_tab.at[ib], xb, sg, add=True)

    def wait_gather(k, S, X):
        bb, ib, sbm = S
        xb, sx, sg, ss = X
        pltpu.make_async_copy(shared_tab.at[ib], xb, sg).wait()

    def issue_store(k, X):
        xb, sx, sg, ss = X
        pltpu.async_copy(xb, out_hbm.at[pl.ds(cbase(k), CHUNK)], ss)

    def wait_store(k, X):
        xb, sx, sg, ss = X
        pltpu.make_async_copy(xb, out_hbm.at[pl.ds(cbase(k), CHUNK)],
                              ss).wait()

    @pl.when(wid == NW - 1)
    def _():
        tb = N_FULL * CHUNK
        pltpu.sync_copy(batch_hbm.at[pl.ds(tb, TAIL)], bb0.at[pl.ds(0, TAIL)])
        pltpu.sync_copy(x_hbm.at[pl.ds(tb, TAIL)], xb0.at[pl.ds(0, TAIL)])
        for j in range(TAIL // L):
            vb = bb0[pl.ds(L * j, L)]
            st = plsc.load_gather(starts_v, [vb])
            pos = tb + L * j + iota16 - st + 1
            ib0[pl.ds(L * j, L)] = jnp.minimum(pos, TABLE_ROWS - 1)
        pltpu.async_copy(shared_tab.at[ib0.at[pl.ds(0, TAIL)]],
                         xb0.at[pl.ds(0, TAIL)], sg0, add=True).wait()
        pltpu.sync_copy(xb0.at[pl.ds(0, TAIL)], out_hbm.at[pl.ds(tb, TAIL)])

    n = nchunks

    issue_loads(0, SB[0], XB[0])
    issue_loads(1, SB[1], XB[1])
    compute_idx(0, SB[0])
    issue_gather_add(0, SB[0], XB[0])

    def iter_body(k, b2, b4):
        wait_gather(k, SB[b2], XB[b4])
        issue_store(k, XB[b4])

        @pl.when(k + 2 < n)
        def _():
            @pl.when(k >= 2)
            def _():
                wait_store(k - 2, XB[(b4 + 2) % NXB])
            issue_loads(k + 2, SB[b2], XB[(b4 + 2) % NXB])

        @pl.when(k + 1 < n)
        def _():
            compute_idx(k + 1, SB[1 - b2])
            issue_gather_add(k + 1, SB[1 - b2], XB[(b4 + 1) % NXB])

    def loop_body(k4, carry):
        for off in range(NXB):
            k = NXB * k4 + off

            @pl.when(k < n)
            def _():
                iter_body(k, off % 2, off)

        return carry

    lax.fori_loop(0, (PER_W + NXB) // NXB, loop_body, 0)

    @pl.when(wid < REM)
    def _():
        for k in range(PER_W - 3, PER_W + 1):
            wait_store(k, XB[k % NXB])

    @pl.when(wid >= REM)
    def _():
        for k in range(PER_W - 4, PER_W):
            wait_store(k, XB[k % NXB])


def kernel(x, batch, table):
    mesh = plsc.VectorSubcoreMesh(core_axis_name="c", subcore_axis_name="s")
    f = pl.kernel(
        _body,
        out_type=jax.ShapeDtypeStruct((N, DIM), jnp.float32),
        mesh=mesh,
        compiler_params=pltpu.CompilerParams(needs_layout_passes=False),
        scratch_types=[
            pltpu.VMEM((SCAN + L,), jnp.int32),
            pltpu.VMEM((64,), jnp.int32),
            pltpu.VMEM((64,), jnp.int32),
            pltpu.VMEM((64,), jnp.int32),
            pltpu.VMEM((64,), jnp.int32),
            pltpu.VMEM_SHARED((64,), jnp.int32),
            pltpu.VMEM_SHARED((TABLE_ROWS, DIM), jnp.float32),
            pltpu.VMEM((CHUNK,), jnp.int32),
            pltpu.VMEM((CHUNK,), jnp.int32),
            pltpu.VMEM((CHUNK,), jnp.int32),
            pltpu.VMEM((CHUNK,), jnp.int32),
            pltpu.VMEM((CHUNK, DIM), jnp.float32),
            pltpu.VMEM((CHUNK, DIM), jnp.float32),
            pltpu.VMEM((CHUNK, DIM), jnp.float32),
            pltpu.VMEM((CHUNK, DIM), jnp.float32),
            pltpu.SemaphoreType.DMA,
            pltpu.SemaphoreType.DMA,
            pltpu.SemaphoreType.DMA,
            pltpu.SemaphoreType.DMA,
            pltpu.SemaphoreType.DMA,
            pltpu.SemaphoreType.DMA,
            pltpu.SemaphoreType.DMA,
            pltpu.SemaphoreType.DMA,
            pltpu.SemaphoreType.DMA,
            pltpu.SemaphoreType.DMA,
            pltpu.SemaphoreType.DMA,
            pltpu.SemaphoreType.DMA,
            pltpu.SemaphoreType.DMA,
            pltpu.SemaphoreType.DMA,
            pltpu.SemaphoreType.DMA,
        ],
    )
    return f(x, batch.astype(jnp.int32), table)
